# baseline (device time: 100384 ns/iter reference)
import jax
import jax.numpy as jnp
from jax import lax
from jax.experimental import pallas as pl
from jax.experimental.pallas import tpu as pltpu

N_DEV = 4
N_SUB = 4


def kernel(x, w_mat):
    m, k = x.shape
    _, n = w_mat.shape
    m_per = m // N_DEV
    n_half = n // 2
    n_sub = n_half // N_SUB

    def body(x_ref, w_ref, out_ref, xb_ref, wb_ref, commR_ref, commL_ref,
             amax_send_ref, amax_recv_ref,
             sendR_sems, recvR_sems, sendL_sems, recvL_sems,
             amax_send_sems, amax_recv_sems):
        my = lax.axis_index("i")
        left = (my - 1) % N_DEV
        right = (my + 1) % N_DEV


        def part(c, col0):
            return jnp.dot(
                xb_ref[pl.ds(c * m_per, m_per), :],
                wb_ref[:, col0:col0 + n_sub],
                preferred_element_type=jnp.float32,
            )

        def partR(c, sub):
            return part(c, sub * n_sub)

        def partL(c, sub):
            return part(c, n_half + sub * n_sub)

        def make_rdma(comm_ref, send_sems, recv_sems, s, sub, dev):
            return pltpu.make_async_remote_copy(
                src_ref=comm_ref.at[s % 3, sub],
                dst_ref=comm_ref.at[(s + 1) % 3, sub],
                send_sem=send_sems.at[s % 3, sub],
                recv_sem=recv_sems.at[(s + 1) % 3, sub],
                device_id=(dev,),
                device_id_type=pl.DeviceIdType.MESH,
            )

        def mkR(s, sub):
            return make_rdma(commR_ref, sendR_sems, recvR_sems, s, sub, right)

        def mkL(s, sub):
            return make_rdma(commL_ref, sendL_sems, recvL_sems, s, sub, left)

        barrier_sem = pltpu.get_barrier_semaphore()
        for nbr in (left, right):
            pl.semaphore_signal(
                barrier_sem, inc=1,
                device_id=(nbr,), device_id_type=pl.DeviceIdType.MESH,
            )

        xb_ref[...] = x_ref[...].astype(jnp.bfloat16)
        wb_ref[...] = w_ref[...].astype(jnp.bfloat16)

        rdmaR = [None] * N_SUB
        rdmaL = [None] * N_SUB
        for sub in range(N_SUB):
            commR_ref[0, sub] = partR((my - 1) % N_DEV, sub).astype(jnp.bfloat16)
            commL_ref[0, sub] = partL((my + 1) % N_DEV, sub).astype(jnp.bfloat16)

        pl.semaphore_wait(barrier_sem, 2)

        all_sends = []
        for sub in range(N_SUB):
            rdmaR[sub] = mkR(0, sub)
            rdmaR[sub].start()
            rdmaL[sub] = mkL(0, sub)
            rdmaL[sub].start()
        all_sends += rdmaR + rdmaL

        for s in (1, 2):
            pR = [partR((my - 1 - s) % N_DEV, sub) for sub in range(N_SUB)]
            pL = [partL((my + 1 + s) % N_DEV, sub) for sub in range(N_SUB)]
            nrR = [None] * N_SUB
            nrL = [None] * N_SUB
            for sub in range(N_SUB):
                rdmaR[sub].wait_recv()
                commR_ref[s % 3, sub] = (
                    pR[sub] + commR_ref[s % 3, sub].astype(jnp.float32)
                ).astype(jnp.bfloat16)
                nrR[sub] = mkR(s, sub)
                nrR[sub].start()
                rdmaL[sub].wait_recv()
                commL_ref[s % 3, sub] = (
                    pL[sub] + commL_ref[s % 3, sub].astype(jnp.float32)
                ).astype(jnp.bfloat16)
                nrL[sub] = mkL(s, sub)
                nrL[sub].start()
            rdmaR, rdmaL = nrR, nrL
            all_sends += rdmaR + rdmaL

        pR = [partR(my, sub) for sub in range(N_SUB)]
        pL = [partL(my, sub) for sub in range(N_SUB)]
        y = [None] * (2 * N_SUB)
        local_amax = jnp.float32(0.0)
        for sub in range(N_SUB):
            rdmaR[sub].wait_recv()
            y[sub] = pR[sub] + commR_ref[0, sub].astype(jnp.float32)
            local_amax = jnp.maximum(local_amax, jnp.max(jnp.abs(y[sub])))
            rdmaL[sub].wait_recv()
            y[N_SUB + sub] = pL[sub] + commL_ref[0, sub].astype(jnp.float32)
            local_amax = jnp.maximum(
                local_amax, jnp.max(jnp.abs(y[N_SUB + sub])))

        amax_send_ref[...] = jnp.full((8, 128), local_amax, jnp.float32)
        amax_recv_ref[pl.ds(my, 1)] = amax_send_ref[...][None]
        for off in range(1, N_DEV):
            tgt = (my + off) % N_DEV
            pltpu.make_async_remote_copy(
                src_ref=amax_send_ref,
                dst_ref=amax_recv_ref.at[my],
                send_sem=amax_send_sems.at[off - 1],
                recv_sem=amax_recv_sems.at[my],
                device_id=(tgt,),
                device_id_type=pl.DeviceIdType.MESH,
            ).start()
        for off in range(1, N_DEV):
            src_dev = (my + off) % N_DEV
            pltpu.make_async_remote_copy(
                src_ref=amax_send_ref,
                dst_ref=amax_recv_ref.at[src_dev],
                send_sem=amax_send_sems.at[0],
                recv_sem=amax_recv_sems.at[src_dev],
                device_id=(src_dev,),
                device_id_type=pl.DeviceIdType.MESH,
            ).wait_recv()

        amax_g = jnp.max(amax_recv_ref[...])
        scale = amax_g / 127.0
        for i in range(2 * N_SUB):
            q = jnp.clip(jnp.round(y[i] / scale), -127.0, 127.0)
            out_ref[:, i * n_sub:(i + 1) * n_sub] = (
                q * scale).astype(jnp.bfloat16)

        for rdma in all_sends:
            rdma.wait_send()
        for off in range(1, N_DEV):
            pltpu.make_async_remote_copy(
                src_ref=amax_send_ref,
                dst_ref=amax_recv_ref.at[my],
                send_sem=amax_send_sems.at[off - 1],
                recv_sem=amax_recv_sems.at[my],
                device_id=(my,),
                device_id_type=pl.DeviceIdType.MESH,
            ).wait_send()

    return pl.pallas_call(
        body,
        out_shape=jax.ShapeDtypeStruct((m_per, n), jnp.bfloat16),
        in_specs=[
            pl.BlockSpec(memory_space=pltpu.VMEM),
            pl.BlockSpec(memory_space=pltpu.VMEM),
        ],
        out_specs=pl.BlockSpec(memory_space=pltpu.VMEM),
        scratch_shapes=[
            pltpu.VMEM((m, k), jnp.bfloat16),
            pltpu.VMEM((k, n), jnp.bfloat16),
            pltpu.VMEM((3, N_SUB, m_per, n_sub), jnp.bfloat16),
            pltpu.VMEM((3, N_SUB, m_per, n_sub), jnp.bfloat16),
            pltpu.VMEM((8, 128), jnp.float32),
            pltpu.VMEM((N_DEV, 8, 128), jnp.float32),
            pltpu.SemaphoreType.DMA((3, N_SUB)),
            pltpu.SemaphoreType.DMA((3, N_SUB)),
            pltpu.SemaphoreType.DMA((3, N_SUB)),
            pltpu.SemaphoreType.DMA((3, N_SUB)),
            pltpu.SemaphoreType.DMA((N_DEV - 1,)),
            pltpu.SemaphoreType.DMA((N_DEV,)),
        ],
        compiler_params=pltpu.CompilerParams(
            collective_id=0,
            vmem_limit_bytes=100 * 1024 * 1024,
        ),
    )(x, w_mat)


# device time: 98359 ns/iter; 1.0206x vs baseline; 1.0206x over previous
import jax
import jax.numpy as jnp
from jax import lax
from jax.experimental import pallas as pl
from jax.experimental.pallas import tpu as pltpu

N_DEV = 4
N_SUB = 4


def kernel(x, w_mat):
    m, k = x.shape
    _, n = w_mat.shape
    m_per = m // N_DEV
    n_half = n // 2
    n_sub = n_half // N_SUB

    def body(x_ref, w_ref, out_ref, xb_ref, wb_ref, commR_ref, commL_ref,
             amax_send_ref, amax_recv_ref,
             sendR_sems, recvR_sems, sendL_sems, recvL_sems,
             amax_send_sems, amax_recv_sems):
        my = lax.axis_index("i")
        left = (my - 1) % N_DEV
        right = (my + 1) % N_DEV

        def part(c, col0):
            return jnp.dot(
                xb_ref[pl.ds(c * m_per, m_per), :],
                wb_ref[:, col0:col0 + n_sub],
                preferred_element_type=jnp.float32,
            )

        def partR(c, sub):
            return part(c, sub * n_sub)

        def partL(c, sub):
            return part(c, n_half + sub * n_sub)

        def make_rdma(comm_ref, send_sems, recv_sems, s, sub, dev):
            return pltpu.make_async_remote_copy(
                src_ref=comm_ref.at[s % 3, sub],
                dst_ref=comm_ref.at[(s + 1) % 3, sub],
                send_sem=send_sems.at[s % 3, sub],
                recv_sem=recv_sems.at[(s + 1) % 3, sub],
                device_id=(dev,),
                device_id_type=pl.DeviceIdType.MESH,
            )

        def mkR(s, sub):
            return make_rdma(commR_ref, sendR_sems, recvR_sems, s, sub, right)

        def mkL(s, sub):
            return make_rdma(commL_ref, sendL_sems, recvL_sems, s, sub, left)

        xb_ref[...] = x_ref[...].astype(jnp.bfloat16)
        wb_ref[...] = w_ref[...].astype(jnp.bfloat16)

        barrier_sem = pltpu.get_barrier_semaphore()
        for nbr in (left, right):
            pl.semaphore_signal(
                barrier_sem, inc=1,
                device_id=(nbr,), device_id_type=pl.DeviceIdType.MESH,
            )
        pl.semaphore_wait(barrier_sem, 2)

        all_sends = []

        rdmaR = [None] * N_SUB
        rdmaL = [None] * N_SUB
        for sub in range(N_SUB):
            commR_ref[0, sub] = partR((my - 1) % N_DEV, sub).astype(jnp.bfloat16)
            rdmaR[sub] = mkR(0, sub)
            rdmaR[sub].start()
            commL_ref[0, sub] = partL((my + 1) % N_DEV, sub).astype(jnp.bfloat16)
            rdmaL[sub] = mkL(0, sub)
            rdmaL[sub].start()
        all_sends += rdmaR + rdmaL

        for s in (1, 2):
            pR = [partR((my - 1 - s) % N_DEV, sub) for sub in range(N_SUB)]
            pL = [partL((my + 1 + s) % N_DEV, sub) for sub in range(N_SUB)]
            nrR = [None] * N_SUB
            nrL = [None] * N_SUB
            for sub in range(N_SUB):
                rdmaR[sub].wait_recv()
                commR_ref[s % 3, sub] = (
                    pR[sub] + commR_ref[s % 3, sub].astype(jnp.float32)
                ).astype(jnp.bfloat16)
                nrR[sub] = mkR(s, sub)
                nrR[sub].start()
                rdmaL[sub].wait_recv()
                commL_ref[s % 3, sub] = (
                    pL[sub] + commL_ref[s % 3, sub].astype(jnp.float32)
                ).astype(jnp.bfloat16)
                nrL[sub] = mkL(s, sub)
                nrL[sub].start()
            rdmaR, rdmaL = nrR, nrL
            all_sends += rdmaR + rdmaL

        pR = [partR(my, sub) for sub in range(N_SUB)]
        pL = [partL(my, sub) for sub in range(N_SUB)]
        y = [None] * (2 * N_SUB)
        local_amax = jnp.float32(0.0)
        for sub in range(N_SUB):
            rdmaR[sub].wait_recv()
            y[sub] = pR[sub] + commR_ref[0, sub].astype(jnp.float32)
            local_amax = jnp.maximum(local_amax, jnp.max(jnp.abs(y[sub])))
            rdmaL[sub].wait_recv()
            y[N_SUB + sub] = pL[sub] + commL_ref[0, sub].astype(jnp.float32)
            local_amax = jnp.maximum(
                local_amax, jnp.max(jnp.abs(y[N_SUB + sub])))

        amax_send_ref[...] = jnp.full((8, 128), local_amax, jnp.float32)
        amax_recv_ref[pl.ds(my, 1)] = amax_send_ref[...][None]
        for off in range(1, N_DEV):
            tgt = (my + off) % N_DEV
            pltpu.make_async_remote_copy(
                src_ref=amax_send_ref,
                dst_ref=amax_recv_ref.at[my],
                send_sem=amax_send_sems.at[off - 1],
                recv_sem=amax_recv_sems.at[my],
                device_id=(tgt,),
                device_id_type=pl.DeviceIdType.MESH,
            ).start()
        for off in range(1, N_DEV):
            src_dev = (my + off) % N_DEV
            pltpu.make_async_remote_copy(
                src_ref=amax_send_ref,
                dst_ref=amax_recv_ref.at[src_dev],
                send_sem=amax_send_sems.at[0],
                recv_sem=amax_recv_sems.at[src_dev],
                device_id=(src_dev,),
                device_id_type=pl.DeviceIdType.MESH,
            ).wait_recv()

        amax_g = jnp.max(amax_recv_ref[...])
        scale = amax_g / 127.0
        for i in range(2 * N_SUB):
            q = jnp.clip(jnp.round(y[i] / scale), -127.0, 127.0)
            out_ref[:, i * n_sub:(i + 1) * n_sub] = (
                q * scale).astype(jnp.bfloat16)

        for rdma in all_sends:
            rdma.wait_send()
        for off in range(1, N_DEV):
            pltpu.make_async_remote_copy(
                src_ref=amax_send_ref,
                dst_ref=amax_recv_ref.at[my],
                send_sem=amax_send_sems.at[off - 1],
                recv_sem=amax_recv_sems.at[my],
                device_id=(my,),
                device_id_type=pl.DeviceIdType.MESH,
            ).wait_send()

    return pl.pallas_call(
        body,
        out_shape=jax.ShapeDtypeStruct((m_per, n), jnp.bfloat16),
        in_specs=[
            pl.BlockSpec(memory_space=pltpu.VMEM),
            pl.BlockSpec(memory_space=pltpu.VMEM),
        ],
        out_specs=pl.BlockSpec(memory_space=pltpu.VMEM),
        scratch_shapes=[
            pltpu.VMEM((m, k), jnp.bfloat16),
            pltpu.VMEM((k, n), jnp.bfloat16),
            pltpu.VMEM((3, N_SUB, m_per, n_sub), jnp.bfloat16),
            pltpu.VMEM((3, N_SUB, m_per, n_sub), jnp.bfloat16),
            pltpu.VMEM((8, 128), jnp.float32),
            pltpu.VMEM((N_DEV, 8, 128), jnp.float32),
            pltpu.SemaphoreType.DMA((3, N_SUB)),
            pltpu.SemaphoreType.DMA((3, N_SUB)),
            pltpu.SemaphoreType.DMA((3, N_SUB)),
            pltpu.SemaphoreType.DMA((3, N_SUB)),
            pltpu.SemaphoreType.DMA((N_DEV - 1,)),
            pltpu.SemaphoreType.DMA((N_DEV,)),
        ],
        compiler_params=pltpu.CompilerParams(
            collective_id=0,
            vmem_limit_bytes=100 * 1024 * 1024,
        ),
    )(x, w_mat)
